# Initial kernel scaffold; baseline (speedup 1.0000x reference)
import jax
import jax.numpy as jnp
from jax.experimental import pallas as pl


def kernel(hist_ids, wish_ids, hist_dense, wish_dense, cand_ids, cand_dense, author_table, lang_table, tag_table, u_p0, u_p1, u_p2, u_p3, u_p4, u_p5, i_p0, i_p1, i_p2, i_p3, i_p4, i_p5):
    raise NotImplementedError("write your pallas kernel here")


# speedup vs baseline: 2.1820x; 2.1820x over previous
import functools
import jax
import jax.numpy as jnp
from jax import lax
from jax.experimental import pallas as pl
from jax.experimental.pallas import tpu as pltpu
from jax.experimental.pallas import tpu_sc as plsc

B = 1024
L = 50
E = 64
DD = 4 * E + 3
H1 = 512

NW = 32
IDS_TOTAL = 2 * B * L + B
CHUNK = 128
NCHUNK = -(-IDS_TOTAL // (NW * CHUNK))
IDS_PAD = NW * CHUNK * NCHUNK
PER_W = CHUNK * NCHUNK

OFF_WISH = L * 3 * E
OFF_HD = 2 * L * 3 * E
OFF_WD = OFF_HD + L * DD


def _sc_gather_body(ids, at, lt, tt, oa, ol, ot,
                    idx_v, a_v, l_v, t_v, sem_a, sem_l, sem_t):
    wid = lax.axis_index("s") * 2 + lax.axis_index("c")
    base = wid * PER_W

    def step(i, carry):
        off = base + i * CHUNK
        pltpu.sync_copy(ids.at[pl.ds(off, CHUNK)], idx_v)
        ca = pltpu.async_copy(at.at[idx_v], a_v, sem_a)
        cl = pltpu.async_copy(lt.at[idx_v], l_v, sem_l)
        ct = pltpu.async_copy(tt.at[idx_v], t_v, sem_t)
        ca.wait()
        cl.wait()
        ct.wait()
        pltpu.sync_copy(a_v, oa.at[pl.ds(off, CHUNK)])
        pltpu.sync_copy(l_v, ol.at[pl.ds(off, CHUNK)])
        pltpu.sync_copy(t_v, ot.at[pl.ds(off, CHUNK)])
        return carry

    lax.fori_loop(0, NCHUNK, step, 0)


@functools.cache
def _make_sc_gather():
    return functools.partial(
        pl.kernel,
        out_type=(
            jax.ShapeDtypeStruct((IDS_PAD, E), jnp.float32),
            jax.ShapeDtypeStruct((IDS_PAD, E), jnp.float32),
            jax.ShapeDtypeStruct((IDS_PAD, E), jnp.float32),
        ),
        mesh=plsc.VectorSubcoreMesh(core_axis_name="c", subcore_axis_name="s"),
        compiler_params=pltpu.CompilerParams(use_tc_tiling_on_sc=False),
        scratch_types=(
            pltpu.VMEM((CHUNK,), jnp.int32),
            pltpu.VMEM((CHUNK, E), jnp.float32),
            pltpu.VMEM((CHUNK, E), jnp.float32),
            pltpu.VMEM((CHUNK, E), jnp.float32),
            pltpu.SemaphoreType.DMA,
            pltpu.SemaphoreType.DMA,
            pltpu.SemaphoreType.DMA,
        ),
    )(_sc_gather_body)


WIN = 272


def _tc1_body(ah, lh, th, aw, lw, tw, hd, wd, wh, ww, w0any, b0, wtail, out,
              acc, whd, wwd, sem_hd, sem_wd):
    l = pl.program_id(0)

    def aligned(off):
        return pl.multiple_of((off // 8) * 8, 8)

    def start_dma(step, slot):
        h_off = OFF_HD + step * DD
        pltpu.make_async_copy(
            w0any.at[pl.ds(aligned(h_off), WIN), :],
            whd.at[slot], sem_hd.at[slot]).start()

        @pl.when(step < L - 1)
        def _():
            w_off = OFF_WD + step * DD
            pltpu.make_async_copy(
                w0any.at[pl.ds(aligned(w_off), WIN), :],
                wwd.at[slot], sem_wd.at[slot]).start()

    @pl.when(l == 0)
    def _():
        start_dma(0, 0)

    @pl.when(l + 1 < L)
    def _():
        start_dma(l + 1, (l + 1) % 2)

    slot = l % 2
    h_off = OFF_HD + l * DD
    pltpu.make_async_copy(
        w0any.at[pl.ds(aligned(h_off), WIN), :],
        whd.at[slot], sem_hd.at[slot]).wait()

    @pl.when(l < L - 1)
    def _():
        w_off = OFF_WD + l * DD
        pltpu.make_async_copy(
            w0any.at[pl.ds(aligned(w_off), WIN), :],
            wwd.at[slot], sem_wd.at[slot]).wait()

    f32 = jnp.float32
    a = jnp.dot(ah[...], wh[0:E, :], preferred_element_type=f32)
    a += jnp.dot(lh[...], wh[E:2 * E, :], preferred_element_type=f32)
    a += jnp.dot(th[...], wh[2 * E:3 * E, :], preferred_element_type=f32)
    a += jnp.dot(aw[...], ww[0:E, :], preferred_element_type=f32)
    a += jnp.dot(lw[...], ww[E:2 * E, :], preferred_element_type=f32)
    a += jnp.dot(tw[...], ww[2 * E:3 * E, :], preferred_element_type=f32)
    w_h = pltpu.roll(whd[slot], WIN - h_off % 8, axis=0)
    a += jnp.dot(hd[:, 0, 0, :], w_h[0:DD, :], preferred_element_type=f32)

    @pl.when(l == 0)
    def _():
        acc[...] = a

    @pl.when(l > 0)
    def _():
        acc[...] += a

    @pl.when(l < L - 1)
    def _():
        w_off = OFF_WD + l * DD
        w_w = pltpu.roll(wwd[slot], WIN - w_off % 8, axis=0)
        acc[...] += jnp.dot(wd[:, 0, 0, :], w_w[0:DD, :],
                            preferred_element_type=f32)

    @pl.when(l == L - 1)
    def _():
        acc[...] += jnp.dot(wd[:, 0, 0, :], wtail[...],
                            preferred_element_type=f32)
        out[...] = jnp.maximum(acc[...] + b0[...], 0.0)


def _tc1(oa, ol, ot, hd, wd, w0, b0, wtail):
    emb = pl.BlockSpec((B, E), lambda l: (l, 0))
    embw = pl.BlockSpec((B, E), lambda l: (L + l, 0))
    dense = pl.BlockSpec((B, 1, 1, DD), lambda l: (0, l, 0, 0))
    return pl.pallas_call(
        _tc1_body,
        grid=(L,),
        in_specs=[
            emb, emb, emb, embw, embw, embw, dense, dense,
            pl.BlockSpec((3 * E, H1), lambda l: (l, 0)),
            pl.BlockSpec((3 * E, H1), lambda l: (L + l, 0)),
            pl.BlockSpec(memory_space=pl.ANY),
            pl.BlockSpec((1, H1), lambda l: (0, 0)),
            pl.BlockSpec((DD, H1), lambda l: (0, 0)),
        ],
        out_specs=pl.BlockSpec((B, H1), lambda l: (0, 0)),
        out_shape=jax.ShapeDtypeStruct((B, H1), jnp.float32),
        scratch_shapes=[
            pltpu.VMEM((B, H1), jnp.float32),
            pltpu.VMEM((2, WIN, H1), jnp.float32),
            pltpu.VMEM((2, WIN, H1), jnp.float32),
            pltpu.SemaphoreType.DMA((2,)),
            pltpu.SemaphoreType.DMA((2,)),
        ],
        compiler_params=pltpu.CompilerParams(
            dimension_semantics=("arbitrary",)),
    )(oa, ol, ot, oa, ol, ot, hd, wd, w0, w0, w0, b0, wtail)


def _tc2_body(h1, ca, cl, ct, cd, w1, b1, w2, b2,
              wi0, bi0, wi1, bi1, wi2, bi2, out):
    f32 = jnp.float32
    u = jnp.maximum(jnp.dot(h1[...], w1[...], preferred_element_type=f32)
                    + b1[...], 0.0)
    u = jnp.dot(u, w2[...], preferred_element_type=f32) + b2[...]

    xi = jnp.dot(ca[...], wi0[0:E, :], preferred_element_type=f32)
    xi += jnp.dot(cl[...], wi0[E:2 * E, :], preferred_element_type=f32)
    xi += jnp.dot(ct[...], wi0[2 * E:3 * E, :], preferred_element_type=f32)
    xi += jnp.dot(cd[...], wi0[3 * E:3 * E + DD, :],
                  preferred_element_type=f32)
    it = jnp.maximum(xi + bi0[...], 0.0)
    it = jnp.maximum(jnp.dot(it, wi1[...], preferred_element_type=f32)
                     + bi1[...], 0.0)
    it = jnp.dot(it, wi2[...], preferred_element_type=f32) + bi2[...]

    out[...] = jnp.sum(u * it, axis=1, keepdims=True)


def _tc2(h1, oa, ol, ot, cd, w1, b1, w2, b2, wi0, bi0, wi1, bi1, wi2, bi2):
    cand = pl.BlockSpec((B, E), lambda i: (2 * L, 0))
    in_specs = [
        pl.BlockSpec((B, H1), lambda i: (0, 0)),
        cand, cand, cand,
        pl.BlockSpec((B, DD), lambda i: (0, 0)),
    ] + [pl.BlockSpec(w.shape, lambda i: tuple(0 for _ in w.shape))
         for w in (w1, b1, w2, b2, wi0, bi0, wi1, bi1, wi2, bi2)]
    return pl.pallas_call(
        _tc2_body,
        grid=(1,),
        in_specs=in_specs,
        out_specs=pl.BlockSpec((B, 1), lambda i: (0, 0)),
        out_shape=jax.ShapeDtypeStruct((B, 1), jnp.float32),
    )(h1, oa, ol, ot, cd, w1, b1, w2, b2, wi0, bi0, wi1, bi1, wi2, bi2)


def kernel(hist_ids, wish_ids, hist_dense, wish_dense, cand_ids, cand_dense,
           author_table, lang_table, tag_table,
           u_p0, u_p1, u_p2, u_p3, u_p4, u_p5,
           i_p0, i_p1, i_p2, i_p3, i_p4, i_p5):
    ids = jnp.concatenate([
        hist_ids.T.reshape(-1),
        wish_ids.T.reshape(-1),
        cand_ids,
        jnp.zeros((IDS_PAD - IDS_TOTAL,), hist_ids.dtype),
    ]).astype(jnp.int32)

    oa, ol, ot = _make_sc_gather()(ids, author_table, lang_table, tag_table)

    h1 = _tc1(oa, ol, ot,
              hist_dense.reshape(B, L, 1, DD),
              wish_dense.reshape(B, L, 1, DD),
              u_p0, u_p1.reshape(1, H1),
              u_p0[OFF_WD + (L - 1) * DD:])

    return _tc2(h1, oa, ol, ot, cand_dense,
                u_p2, u_p3.reshape(1, -1), u_p4, u_p5.reshape(1, -1),
                i_p0, i_p1.reshape(1, -1), i_p2, i_p3.reshape(1, -1),
                i_p4, i_p5.reshape(1, -1))


# speedup vs baseline: 2.8612x; 1.3113x over previous
import functools
import jax
import jax.numpy as jnp
from jax import lax
from jax.experimental import pallas as pl
from jax.experimental.pallas import tpu as pltpu
from jax.experimental.pallas import tpu_sc as plsc

B = 1024
L = 50
E = 64
DD = 4 * E + 3
H1 = 512

NW = 32
IDS_TOTAL = 2 * B * L + B
CHUNK = 128
NCHUNK = -(-IDS_TOTAL // (NW * CHUNK))
IDS_PAD = NW * CHUNK * NCHUNK
PER_W = CHUNK * NCHUNK

OFF_WISH = L * 3 * E
OFF_HD = 2 * L * 3 * E
OFF_WD = OFF_HD + L * DD


def _sc_gather_body(ids, at, lt, tt, oat,
                    idx_v, a_v, l_v, t_v, sem_a, sem_l, sem_t):
    wid = lax.axis_index("s") * 2 + lax.axis_index("c")
    base = wid * PER_W

    def step(i, carry):
        off = base + i * CHUNK
        pltpu.sync_copy(ids.at[pl.ds(off, CHUNK)], idx_v)
        ca = pltpu.async_copy(at.at[idx_v], a_v, sem_a)
        cl = pltpu.async_copy(lt.at[idx_v], l_v, sem_l)
        ct = pltpu.async_copy(tt.at[idx_v], t_v, sem_t)
        ca.wait()
        cl.wait()
        ct.wait()
        pltpu.sync_copy(a_v, oat.at[pl.ds(off, CHUNK), pl.ds(0, E)])
        pltpu.sync_copy(l_v, oat.at[pl.ds(off, CHUNK), pl.ds(E, E)])
        pltpu.sync_copy(t_v, oat.at[pl.ds(off, CHUNK), pl.ds(2 * E, E)])
        return carry

    lax.fori_loop(0, NCHUNK, step, 0)


@functools.cache
def _make_sc_gather():
    return functools.partial(
        pl.kernel,
        out_type=jax.ShapeDtypeStruct((IDS_PAD, 4 * E), jnp.float32),
        mesh=plsc.VectorSubcoreMesh(core_axis_name="c", subcore_axis_name="s"),
        compiler_params=pltpu.CompilerParams(use_tc_tiling_on_sc=False),
        scratch_types=(
            pltpu.VMEM((CHUNK,), jnp.int32),
            pltpu.VMEM((CHUNK, E), jnp.float32),
            pltpu.VMEM((CHUNK, E), jnp.float32),
            pltpu.VMEM((CHUNK, E), jnp.float32),
            pltpu.SemaphoreType.DMA,
            pltpu.SemaphoreType.DMA,
            pltpu.SemaphoreType.DMA,
        ),
    )(_sc_gather_body)


WIN = 272


def _tc1_body(eh, ew, hd, wd, wh, ww, w0any, b0, wtail, out,
              acc, whd, wwd, sem_hd, sem_wd):
    l = pl.program_id(0)

    def aligned(off):
        return pl.multiple_of((off // 8) * 8, 8)

    def start_dma(step, slot):
        h_off = OFF_HD + step * DD
        pltpu.make_async_copy(
            w0any.at[pl.ds(aligned(h_off), WIN), :],
            whd.at[slot], sem_hd.at[slot]).start()

        @pl.when(step < L - 1)
        def _():
            w_off = OFF_WD + step * DD
            pltpu.make_async_copy(
                w0any.at[pl.ds(aligned(w_off), WIN), :],
                wwd.at[slot], sem_wd.at[slot]).start()

    @pl.when(l == 0)
    def _():
        start_dma(0, 0)

    @pl.when(l + 1 < L)
    def _():
        start_dma(l + 1, (l + 1) % 2)

    slot = l % 2
    h_off = OFF_HD + l * DD
    pltpu.make_async_copy(
        w0any.at[pl.ds(aligned(h_off), WIN), :],
        whd.at[slot], sem_hd.at[slot]).wait()

    @pl.when(l < L - 1)
    def _():
        w_off = OFF_WD + l * DD
        pltpu.make_async_copy(
            w0any.at[pl.ds(aligned(w_off), WIN), :],
            wwd.at[slot], sem_wd.at[slot]).wait()

    f32 = jnp.float32
    a = jnp.dot(eh[:, 0:3 * E], wh[...], preferred_element_type=f32)
    a += jnp.dot(ew[:, 0:3 * E], ww[...], preferred_element_type=f32)
    w_h = pltpu.roll(whd[slot], WIN - h_off % 8, axis=0)
    a += jnp.dot(hd[0], w_h[0:DD, :], preferred_element_type=f32)

    @pl.when(l == 0)
    def _():
        acc[...] = a

    @pl.when(l > 0)
    def _():
        acc[...] += a

    @pl.when(l < L - 1)
    def _():
        w_off = OFF_WD + l * DD
        w_w = pltpu.roll(wwd[slot], WIN - w_off % 8, axis=0)
        acc[...] += jnp.dot(wd[0], w_w[0:DD, :],
                            preferred_element_type=f32)

    @pl.when(l == L - 1)
    def _():
        acc[...] += jnp.dot(wd[0], wtail[...],
                            preferred_element_type=f32)
        out[...] = jnp.maximum(acc[...] + b0[...], 0.0)


def _tc1(oat, hdt, wdt, w0, b0, wtail):
    return pl.pallas_call(
        _tc1_body,
        grid=(L,),
        in_specs=[
            pl.BlockSpec((B, 4 * E), lambda l: (l, 0)),
            pl.BlockSpec((B, 4 * E), lambda l: (L + l, 0)),
            pl.BlockSpec((1, B, DD), lambda l: (l, 0, 0)),
            pl.BlockSpec((1, B, DD), lambda l: (l, 0, 0)),
            pl.BlockSpec((3 * E, H1), lambda l: (l, 0)),
            pl.BlockSpec((3 * E, H1), lambda l: (L + l, 0)),
            pl.BlockSpec(memory_space=pl.ANY),
            pl.BlockSpec((1, H1), lambda l: (0, 0)),
            pl.BlockSpec((DD, H1), lambda l: (0, 0)),
        ],
        out_specs=pl.BlockSpec((B, H1), lambda l: (0, 0)),
        out_shape=jax.ShapeDtypeStruct((B, H1), jnp.float32),
        scratch_shapes=[
            pltpu.VMEM((B, H1), jnp.float32),
            pltpu.VMEM((2, WIN, H1), jnp.float32),
            pltpu.VMEM((2, WIN, H1), jnp.float32),
            pltpu.SemaphoreType.DMA((2,)),
            pltpu.SemaphoreType.DMA((2,)),
        ],
        compiler_params=pltpu.CompilerParams(
            dimension_semantics=("arbitrary",)),
    )(oat, oat, hdt, wdt, w0, w0, w0, b0, wtail)


def _tc2_body(h1, ce, cd, w1, b1, w2, b2,
              wi0, bi0, wi1, bi1, wi2, bi2, out):
    f32 = jnp.float32
    u = jnp.maximum(jnp.dot(h1[...], w1[...], preferred_element_type=f32)
                    + b1[...], 0.0)
    u = jnp.dot(u, w2[...], preferred_element_type=f32) + b2[...]

    xi = jnp.dot(ce[:, 0:3 * E], wi0[0:3 * E, :], preferred_element_type=f32)
    xi += jnp.dot(cd[...], wi0[3 * E:3 * E + DD, :],
                  preferred_element_type=f32)
    it = jnp.maximum(xi + bi0[...], 0.0)
    it = jnp.maximum(jnp.dot(it, wi1[...], preferred_element_type=f32)
                     + bi1[...], 0.0)
    it = jnp.dot(it, wi2[...], preferred_element_type=f32) + bi2[...]

    out[...] = jnp.sum(u * it, axis=1, keepdims=True)


def _tc2(h1, oat, cd, w1, b1, w2, b2, wi0, bi0, wi1, bi1, wi2, bi2):
    in_specs = [
        pl.BlockSpec((B, H1), lambda i: (0, 0)),
        pl.BlockSpec((B, 4 * E), lambda i: (2 * L, 0)),
        pl.BlockSpec((B, DD), lambda i: (0, 0)),
    ] + [pl.BlockSpec(w.shape, lambda i: tuple(0 for _ in w.shape))
         for w in (w1, b1, w2, b2, wi0, bi0, wi1, bi1, wi2, bi2)]
    return pl.pallas_call(
        _tc2_body,
        grid=(1,),
        in_specs=in_specs,
        out_specs=pl.BlockSpec((B, 1), lambda i: (0, 0)),
        out_shape=jax.ShapeDtypeStruct((B, 1), jnp.float32),
    )(h1, oat, cd, w1, b1, w2, b2, wi0, bi0, wi1, bi1, wi2, bi2)


def kernel(hist_ids, wish_ids, hist_dense, wish_dense, cand_ids, cand_dense,
           author_table, lang_table, tag_table,
           u_p0, u_p1, u_p2, u_p3, u_p4, u_p5,
           i_p0, i_p1, i_p2, i_p3, i_p4, i_p5):
    ids = jnp.concatenate([
        hist_ids.T.reshape(-1),
        wish_ids.T.reshape(-1),
        cand_ids,
        jnp.zeros((IDS_PAD - IDS_TOTAL,), hist_ids.dtype),
    ]).astype(jnp.int32)

    oat = _make_sc_gather()(ids, author_table, lang_table, tag_table)

    h1 = _tc1(oat,
              jnp.transpose(hist_dense, (1, 0, 2)),
              jnp.transpose(wish_dense, (1, 0, 2)),
              u_p0, u_p1.reshape(1, H1),
              u_p0[OFF_WD + (L - 1) * DD:])

    return _tc2(h1, oat, cand_dense,
                u_p2, u_p3.reshape(1, -1), u_p4, u_p5.reshape(1, -1),
                i_p0, i_p1.reshape(1, -1), i_p2, i_p3.reshape(1, -1),
                i_p4, i_p5.reshape(1, -1))
